# baseline (speedup 1.0000x reference)
import functools

import jax
import jax.numpy as jnp
from jax import lax
from jax.experimental import pallas as pl
from jax.experimental.pallas import tpu as pltpu
from jax.experimental.pallas import tpu_sc as plsc

_N = 10000
_NP = 10240
_E = 160000
_K = 5
_DT = 128
_NS = 16
_EB = 128
_NB = 1280
_EP = _NB * _EB
_BSUB = _NB // _NS
_ECH = _EP // _NS
_N_SUB = _NP // _NS
_ZR = 64


def _sc_mesh():
    return plsc.VectorSubcoreMesh(core_axis_name="c", subcore_axis_name="s")


def _zero_fill(zero_v):
    @pl.loop(0, _ZR)
    def _(rr):
        @pl.loop(0, _DT, step=16)
        def _(j):
            zero_v[rr, pl.ds(j, 16)] = jnp.zeros((16,), jnp.float32)


def _zero_acc(acc_sh, zero_v, s):
    @pl.loop(0, _N_SUB // _ZR)
    def _(zz):
        pltpu.sync_copy(zero_v, acc_sh.at[pl.ds(s * _N_SUB + zz * _ZR, _ZR)])


def _stream_batches(g_hbm, acc_sh, srcf_hbm, dstf_hbm, src_v, dst_v,
                    rows_v, sem, src_base, dst_base, n):
    @pl.loop(0, n)
    def _(k):
        pltpu.sync_copy(srcf_hbm.at[pl.ds(src_base(k), _EB)], src_v)
        pltpu.sync_copy(dstf_hbm.at[pl.ds(dst_base(k), _EB)], dst_v)
        pltpu.async_copy(g_hbm.at[src_v], rows_v, sem).wait()
        pltpu.sync_copy(rows_v, acc_sh.at[dst_v], add=True)


def _prop_scratch():
    return [
        pltpu.VMEM((_EB,), jnp.int32),
        pltpu.VMEM((_EB,), jnp.int32),
        pltpu.VMEM((_EB, _DT), jnp.float32),
        pltpu.VMEM((_ZR, _DT), jnp.float32),
        pltpu.VMEM_SHARED((_NP, _DT), jnp.float32),
        pltpu.SemaphoreType.DMA,
    ]


@functools.partial(
    pl.kernel,
    out_type=jax.ShapeDtypeStruct((2 * _NP, _DT), jnp.float32),
    mesh=_sc_mesh(),
    scratch_types=_prop_scratch(),
)
def _prop2(g_hbm, srcf_hbm, dstf_hbm, out_hbm,
           src_v, dst_v, rows_v, zero_v, acc_sh, sem):
    c = lax.axis_index("c")
    s = lax.axis_index("s")
    _zero_fill(zero_v)
    _zero_acc(acc_sh, zero_v, s)
    plsc.subcore_barrier()
    _stream_batches(g_hbm, acc_sh, srcf_hbm, dstf_hbm, src_v, dst_v,
                    rows_v, sem,
                    lambda k: c * _EP + s * _ECH + k * _EB,
                    lambda k: s * _ECH + k * _EB, _BSUB)
    plsc.subcore_barrier()
    pltpu.sync_copy(acc_sh.at[pl.ds(s * _N_SUB, _N_SUB)],
                    out_hbm.at[pl.ds(c * _NP + s * _N_SUB, _N_SUB)])


@functools.partial(
    pl.kernel,
    out_type=jax.ShapeDtypeStruct((4 * _NP, _DT), jnp.float32),
    mesh=_sc_mesh(),
    scratch_types=_prop_scratch(),
)
def _prop3(g_hbm, srcf_hbm, dstf_hbm, out_hbm,
           src_v, dst_v, rows_v, zero_v, acc_sh, sem):
    c = lax.axis_index("c")
    s = lax.axis_index("s")
    _zero_fill(zero_v)

    _zero_acc(acc_sh, zero_v, s)
    plsc.subcore_barrier()
    _stream_batches(g_hbm, acc_sh, srcf_hbm, dstf_hbm, src_v, dst_v,
                    rows_v, sem,
                    lambda k: c * _EP + s * _ECH + k * _EB,
                    lambda k: s * _ECH + k * _EB, _BSUB)
    plsc.subcore_barrier()
    pltpu.sync_copy(acc_sh.at[pl.ds(s * _N_SUB, _N_SUB)],
                    out_hbm.at[pl.ds(c * _NP + s * _N_SUB, _N_SUB)])

    plsc.subcore_barrier()
    _zero_acc(acc_sh, zero_v, s)
    plsc.subcore_barrier()
    _stream_batches(g_hbm, acc_sh, srcf_hbm, dstf_hbm, src_v, dst_v,
                    rows_v, sem,
                    lambda k: 2 * _EP + s * _ECH + (2 * k + c) * _EB,
                    lambda k: s * _ECH + (2 * k + c) * _EB, _BSUB // 2)
    plsc.subcore_barrier()
    pltpu.sync_copy(acc_sh.at[pl.ds(s * _N_SUB, _N_SUB)],
                    out_hbm.at[pl.ds((2 + c) * _NP + s * _N_SUB, _N_SUB)])


@functools.partial(
    pl.kernel,
    out_type=jax.ShapeDtypeStruct((2 * _NP, _DT), jnp.float32),
    mesh=_sc_mesh(),
    scratch_types=[
        pltpu.VMEM((_BSUB, _EB), jnp.int32),
        pltpu.VMEM((_EB, _DT), jnp.float32),
        pltpu.VMEM((_ZR, _DT), jnp.float32),
        pltpu.VMEM_SHARED((_NP, _DT), jnp.float32),
    ],
)
def _deg_hist(src_hbm, out_hbm, src2_v, ones_v, zero_v, acc_sh):
    c = lax.axis_index("c")
    s = lax.axis_index("s")
    pltpu.sync_copy(src_hbm.at[pl.ds(s * _BSUB, _BSUB)], src2_v)

    @pl.loop(0, _EB)
    def _(rr):
        @pl.loop(0, _DT, step=16)
        def _(j):
            ones_v[rr, pl.ds(j, 16)] = jnp.ones((16,), jnp.float32)

    _zero_fill(zero_v)
    _zero_acc(acc_sh, zero_v, s)
    plsc.subcore_barrier()

    @pl.loop(0, _BSUB // 2)
    def _(i):
        pltpu.sync_copy(ones_v, acc_sh.at[src2_v.at[2 * i + c]], add=True)

    plsc.subcore_barrier()
    pltpu.sync_copy(acc_sh.at[pl.ds(s * _N_SUB, _N_SUB)],
                    out_hbm.at[pl.ds(c * _NP + s * _N_SUB, _N_SUB)])


_BM = 512


def _rec(r, prev, dis_t, a2, with_g, nt, merge):
    nb = _NP // _BM
    grid = (nt * nb,)
    rows = nt * _NP
    ins = [r]
    in_specs = [pl.BlockSpec((_BM, _DT), lambda i: (i, 0))]
    if merge:
        ins.append(r)
        in_specs.append(pl.BlockSpec((_BM, _DT), lambda i: (i + nb, 0)))
    if prev is not None:
        ins.append(prev)
        in_specs.append(pl.BlockSpec((_BM, _DT), lambda i: (i, 0)))
    ins.append(dis_t)
    in_specs.append(pl.BlockSpec((_BM, 1), lambda i: (i, 0)))
    n_out = 2 if with_g else 1
    out_shape = [jax.ShapeDtypeStruct((rows, _DT), jnp.float32)] * n_out
    out_specs = [pl.BlockSpec((_BM, _DT), lambda i: (i, 0))] * n_out

    def body(*refs):
        k = 0
        r_ref = refs[k]; k += 1
        rb_ref = None
        if merge:
            rb_ref = refs[k]; k += 1
        p_ref = None
        if prev is not None:
            p_ref = refs[k]; k += 1
        d_ref = refs[k]; k += 1
        outs = refs[k:]
        rv = r_ref[...]
        if merge:
            in_last = pl.program_id(0) >= (nt - 1) * nb
            rv = rv + jnp.where(in_last, rb_ref[...], 0.0)
        tx = (a2 * d_ref[...]) * rv
        if prev is not None:
            tx = tx - p_ref[...]
        outs[0][...] = tx
        if with_g:
            outs[1][...] = d_ref[...] * tx

    res = pl.pallas_call(body, grid=grid, in_specs=in_specs,
                         out_specs=out_specs, out_shape=out_shape)(*ins)
    return res if with_g else res[0]


def _cheb_out(ts, wt, bias, nt, dout, tiled_out):
    nb = _NP // _BM
    grid = (nb, nt)
    in_specs = [pl.BlockSpec((_BM, _DT), lambda i, c: (c * nb + i, 0))
                for _ in ts]
    in_specs.append(pl.BlockSpec((_K, 1, _DT, dout), lambda i, c: (0, c, 0, 0)))
    in_specs.append(pl.BlockSpec((1, dout), lambda i, c: (0, 0)))
    if tiled_out:
        hd = dout // 2
        out_shape = jax.ShapeDtypeStruct((2, _NP, hd), jnp.float32)
        out_specs = pl.BlockSpec((2, _BM, hd), lambda i, c: (0, i, 0))
    else:
        out_shape = jax.ShapeDtypeStruct((_NP, dout), jnp.float32)
        out_specs = pl.BlockSpec((_BM, dout), lambda i, c: (i, 0))

    def body(*refs):
        t_refs = refs[:_K]
        w_ref, b_ref, out = refs[_K], refs[_K + 1], refs[_K + 2]
        c = pl.program_id(1)
        acc = jnp.dot(t_refs[0][...], w_ref[0, 0],
                      preferred_element_type=jnp.float32)
        for k in range(1, _K):
            acc = acc + jnp.dot(t_refs[k][...], w_ref[k, 0],
                                preferred_element_type=jnp.float32)
        if tiled_out:
            hd2 = dout // 2

            @pl.when(c == 0)
            def _():
                out[0, ...] = acc[:, :hd2]
                out[1, ...] = acc[:, hd2:]

            @pl.when(jnp.logical_and(c > 0, c < nt - 1))
            def _():
                out[0, ...] = out[0, ...] + acc[:, :hd2]
                out[1, ...] = out[1, ...] + acc[:, hd2:]

            @pl.when(c == nt - 1)
            def _():
                out[0, ...] = jnp.maximum(
                    out[0, ...] + acc[:, :hd2] + b_ref[:, :hd2], 0.0)
                out[1, ...] = jnp.maximum(
                    out[1, ...] + acc[:, hd2:] + b_ref[:, hd2:], 0.0)
        else:
            @pl.when(c == 0)
            def _():
                out[...] = acc

            @pl.when(jnp.logical_and(c > 0, c < nt - 1))
            def _():
                out[...] = out[...] + acc

            @pl.when(c == nt - 1)
            def _():
                out[...] = jnp.maximum(out[...] + acc + b_ref[...], 0.0)

    return pl.pallas_call(body, grid=grid, in_specs=in_specs,
                          out_specs=out_specs, out_shape=out_shape)(*ts, wt, bias)


def _layer(xt, dis_t, wt, bias, src, dst, nt, dout, tiled_out):
    prop = _prop3 if nt == 3 else _prop2
    merge = nt == 3
    g0 = _rec(xt, None, dis_t, 1.0, False, nt, False)
    r1 = prop(g0, src, dst)
    tx1, g1 = _rec(r1, None, dis_t, -1.0, True, nt, merge)
    r2 = prop(g1, src, dst)
    tx2, g2 = _rec(r2, xt, dis_t, -2.0, True, nt, merge)
    r3 = prop(g2, src, dst)
    tx3, g3 = _rec(r3, tx1, dis_t, -2.0, True, nt, merge)
    r4 = prop(g3, src, dst)
    tx4 = _rec(r4, tx2, dis_t, -2.0, False, nt, merge)
    return _cheb_out([xt, tx1, tx2, tx3, tx4], wt, bias, nt, dout, tiled_out)


def kernel(x, edge_index, W1, b1, Wl1, bl1, W2, b2, Wl2, bl2):
    pad_e = jnp.full((_EP - _E,), _N, jnp.int32)
    srcf = jnp.concatenate([edge_index[0], pad_e])
    dstf = jnp.concatenate([edge_index[1], pad_e])
    src_t3 = jnp.concatenate([srcf + t * _NP for t in range(3)])
    src_t2 = src_t3[:2 * _EP]

    parts = _deg_hist(srcf.reshape(_NB, _EB))
    deg = parts[:_N, 0] + parts[_NP:_NP + _N, 0]
    dis = jnp.where(deg > 0, deg ** -0.5, 0.0)
    dis_p = jnp.pad(dis, (0, _NP - _N))[:, None]

    x_pad = jnp.pad(x, ((0, _NP - _N), (0, 384 - 350)))
    xt1 = jnp.concatenate(
        [x_pad[:, :128], x_pad[:, 128:256], x_pad[:, 256:]], axis=0)
    dis3 = jnp.concatenate([dis_p, dis_p, dis_p], axis=0)

    w1 = jnp.pad(W1, ((0, 0), (0, 34), (0, 0)))
    w1 = w1.at[0].add(jnp.pad(Wl1, ((0, 34), (0, 0))))
    wt1 = w1.reshape(_K, 3, _DT, 256)
    bb1 = (b1 + bl1)[None, :]

    h_t = _layer(xt1, dis3, wt1, bb1, src_t3, dstf, 3, 256, True)
    xt2 = h_t.reshape(2 * _NP, _DT)
    dis2 = jnp.concatenate([dis_p, dis_p], axis=0)

    w2 = W2.at[0].add(Wl2)
    wt2 = w2.reshape(_K, 2, _DT, _DT)
    bb2 = (b2 + bl2)[None, :]

    out = _layer(xt2, dis2, wt2, bb2, src_t2, dstf, 2, _DT, False)
    return out[:_N]


# speedup vs baseline: 1.5190x; 1.5190x over previous
import functools

import jax
import jax.numpy as jnp
from jax import lax
from jax.experimental import pallas as pl
from jax.experimental.pallas import tpu as pltpu
from jax.experimental.pallas import tpu_sc as plsc

_N = 10000
_NP = 10240
_E = 160000
_K = 5
_DT = 128
_NS = 16
_EB = 128
_NB = 1280
_EP = _NB * _EB
_BSUB = _NB // _NS
_ECH = _EP // _NS
_N_SUB = _NP // _NS
_ZR = 64


def _sc_mesh():
    return plsc.VectorSubcoreMesh(core_axis_name="c", subcore_axis_name="s")


def _zero_fill(zero_v):
    @pl.loop(0, _ZR)
    def _(rr):
        @pl.loop(0, _DT, step=16)
        def _(j):
            zero_v[rr, pl.ds(j, 16)] = jnp.zeros((16,), jnp.float32)


def _zero_acc(acc_sh, zero_v, s):
    @pl.loop(0, _N_SUB // _ZR)
    def _(zz):
        pltpu.sync_copy(zero_v, acc_sh.at[pl.ds(s * _N_SUB + zz * _ZR, _ZR)])


def _stream_batches(g_hbm, acc_sh, srcf_hbm, dstf_hbm, src_v, dst_v,
                    rows_v, sem, src_base, dst_base, n):
    @pl.loop(0, n)
    def _(k):
        pltpu.sync_copy(srcf_hbm.at[pl.ds(src_base(k), _EB)], src_v)
        pltpu.sync_copy(dstf_hbm.at[pl.ds(dst_base(k), _EB)], dst_v)
        pltpu.async_copy(g_hbm.at[src_v], rows_v, sem).wait()
        pltpu.sync_copy(rows_v, acc_sh.at[dst_v], add=True)


def _prop_scratch():
    return [
        pltpu.VMEM((_EB,), jnp.int32),
        pltpu.VMEM((_EB,), jnp.int32),
        pltpu.VMEM((_EB, _DT), jnp.float32),
        pltpu.VMEM((_ZR, _DT), jnp.float32),
        pltpu.VMEM_SHARED((_NP, _DT), jnp.float32),
        pltpu.SemaphoreType.DMA,
    ]


@functools.partial(
    pl.kernel,
    out_type=jax.ShapeDtypeStruct((2 * _NP, _DT), jnp.float32),
    mesh=_sc_mesh(),
    scratch_types=_prop_scratch(),
)
def _prop2(g_hbm, srcf_hbm, dstf_hbm, out_hbm,
           src_v, dst_v, rows_v, zero_v, acc_sh, sem):
    c = lax.axis_index("c")
    s = lax.axis_index("s")
    _zero_fill(zero_v)
    _zero_acc(acc_sh, zero_v, s)
    plsc.subcore_barrier()
    _stream_batches(g_hbm, acc_sh, srcf_hbm, dstf_hbm, src_v, dst_v,
                    rows_v, sem,
                    lambda k: c * _EP + s * _ECH + k * _EB,
                    lambda k: s * _ECH + k * _EB, _BSUB)
    plsc.subcore_barrier()
    pltpu.sync_copy(acc_sh.at[pl.ds(s * _N_SUB, _N_SUB)],
                    out_hbm.at[pl.ds(c * _NP + s * _N_SUB, _N_SUB)])


@functools.partial(
    pl.kernel,
    out_type=jax.ShapeDtypeStruct((4 * _NP, _DT), jnp.float32),
    mesh=_sc_mesh(),
    scratch_types=_prop_scratch(),
)
def _prop3(g_hbm, srcf_hbm, dstf_hbm, out_hbm,
           src_v, dst_v, rows_v, zero_v, acc_sh, sem):
    c = lax.axis_index("c")
    s = lax.axis_index("s")
    _zero_fill(zero_v)

    _zero_acc(acc_sh, zero_v, s)
    plsc.subcore_barrier()
    _stream_batches(g_hbm, acc_sh, srcf_hbm, dstf_hbm, src_v, dst_v,
                    rows_v, sem,
                    lambda k: c * _EP + s * _ECH + k * _EB,
                    lambda k: s * _ECH + k * _EB, _BSUB)
    plsc.subcore_barrier()
    pltpu.sync_copy(acc_sh.at[pl.ds(s * _N_SUB, _N_SUB)],
                    out_hbm.at[pl.ds(c * _NP + s * _N_SUB, _N_SUB)])

    plsc.subcore_barrier()
    _zero_acc(acc_sh, zero_v, s)
    plsc.subcore_barrier()
    _stream_batches(g_hbm, acc_sh, srcf_hbm, dstf_hbm, src_v, dst_v,
                    rows_v, sem,
                    lambda k: 2 * _EP + s * _ECH + (2 * k + c) * _EB,
                    lambda k: s * _ECH + (2 * k + c) * _EB, _BSUB // 2)
    plsc.subcore_barrier()
    pltpu.sync_copy(acc_sh.at[pl.ds(s * _N_SUB, _N_SUB)],
                    out_hbm.at[pl.ds((2 + c) * _NP + s * _N_SUB, _N_SUB)])


@functools.partial(
    pl.kernel,
    out_type=jax.ShapeDtypeStruct((2 * _NP, _DT), jnp.float32),
    mesh=_sc_mesh(),
    scratch_types=[
        pltpu.VMEM((_BSUB, _EB), jnp.int32),
        pltpu.VMEM((_EB, _DT), jnp.float32),
        pltpu.VMEM((_ZR, _DT), jnp.float32),
        pltpu.VMEM_SHARED((_NP, _DT), jnp.float32),
    ],
)
def _deg_hist(src_hbm, out_hbm, src2_v, ones_v, zero_v, acc_sh):
    c = lax.axis_index("c")
    s = lax.axis_index("s")
    pltpu.sync_copy(src_hbm.at[pl.ds(s * _BSUB, _BSUB)], src2_v)

    @pl.loop(0, _EB)
    def _(rr):
        @pl.loop(0, _DT, step=16)
        def _(j):
            ones_v[rr, pl.ds(j, 16)] = jnp.ones((16,), jnp.float32)

    _zero_fill(zero_v)
    _zero_acc(acc_sh, zero_v, s)
    plsc.subcore_barrier()

    @pl.loop(0, _BSUB // 2)
    def _(i):
        pltpu.sync_copy(ones_v, acc_sh.at[src2_v.at[2 * i + c]], add=True)

    plsc.subcore_barrier()
    pltpu.sync_copy(acc_sh.at[pl.ds(s * _N_SUB, _N_SUB)],
                    out_hbm.at[pl.ds(c * _NP + s * _N_SUB, _N_SUB)])


_BM = 512


def _rec(r, prev, dis_t, a2, with_g, nt, merge):
    nb = _NP // _BM
    grid = (nt * nb,)
    rows = nt * _NP
    ins = [r]
    in_specs = [pl.BlockSpec((_BM, _DT), lambda i: (i, 0))]
    if merge:
        ins.append(r)
        in_specs.append(pl.BlockSpec((_BM, _DT), lambda i: (i + nb, 0)))
    if prev is not None:
        ins.append(prev)
        in_specs.append(pl.BlockSpec((_BM, _DT), lambda i: (i, 0)))
    ins.append(dis_t)
    in_specs.append(pl.BlockSpec((_BM, 1), lambda i: (i, 0)))
    n_out = 2 if with_g else 1
    out_shape = [jax.ShapeDtypeStruct((rows, _DT), jnp.float32)] * n_out
    out_specs = [pl.BlockSpec((_BM, _DT), lambda i: (i, 0))] * n_out

    def body(*refs):
        k = 0
        r_ref = refs[k]; k += 1
        rb_ref = None
        if merge:
            rb_ref = refs[k]; k += 1
        p_ref = None
        if prev is not None:
            p_ref = refs[k]; k += 1
        d_ref = refs[k]; k += 1
        outs = refs[k:]
        rv = r_ref[...]
        if merge:
            in_last = pl.program_id(0) >= (nt - 1) * nb
            rv = rv + jnp.where(in_last, rb_ref[...], 0.0)
        tx = (a2 * d_ref[...]) * rv
        if prev is not None:
            tx = tx - p_ref[...]
        outs[0][...] = tx
        if with_g:
            outs[1][...] = d_ref[...] * tx

    res = pl.pallas_call(body, grid=grid, in_specs=in_specs,
                         out_specs=out_specs, out_shape=out_shape)(*ins)
    return res if with_g else res[0]


def _cheb_out(ts, wt, bias, nt, dout, tiled_out):
    nb = _NP // _BM
    grid = (nb, nt)
    in_specs = [pl.BlockSpec((_BM, _DT), lambda i, c: (c * nb + i, 0))
                for _ in ts]
    in_specs.append(pl.BlockSpec((_K, 1, _DT, dout), lambda i, c: (0, c, 0, 0)))
    in_specs.append(pl.BlockSpec((1, dout), lambda i, c: (0, 0)))
    if tiled_out:
        hd = dout // 2
        out_shape = jax.ShapeDtypeStruct((2, _NP, hd), jnp.float32)
        out_specs = pl.BlockSpec((2, _BM, hd), lambda i, c: (0, i, 0))
    else:
        out_shape = jax.ShapeDtypeStruct((_NP, dout), jnp.float32)
        out_specs = pl.BlockSpec((_BM, dout), lambda i, c: (i, 0))

    def body(*refs):
        t_refs = refs[:_K]
        w_ref, b_ref, out = refs[_K], refs[_K + 1], refs[_K + 2]
        c = pl.program_id(1)
        acc = jnp.dot(t_refs[0][...], w_ref[0, 0],
                      preferred_element_type=jnp.float32)
        for k in range(1, _K):
            acc = acc + jnp.dot(t_refs[k][...], w_ref[k, 0],
                                preferred_element_type=jnp.float32)
        if tiled_out:
            hd2 = dout // 2

            @pl.when(c == 0)
            def _():
                out[0, ...] = acc[:, :hd2]
                out[1, ...] = acc[:, hd2:]

            @pl.when(jnp.logical_and(c > 0, c < nt - 1))
            def _():
                out[0, ...] = out[0, ...] + acc[:, :hd2]
                out[1, ...] = out[1, ...] + acc[:, hd2:]

            @pl.when(c == nt - 1)
            def _():
                out[0, ...] = jnp.maximum(
                    out[0, ...] + acc[:, :hd2] + b_ref[:, :hd2], 0.0)
                out[1, ...] = jnp.maximum(
                    out[1, ...] + acc[:, hd2:] + b_ref[:, hd2:], 0.0)
        else:
            @pl.when(c == 0)
            def _():
                out[...] = acc

            @pl.when(jnp.logical_and(c > 0, c < nt - 1))
            def _():
                out[...] = out[...] + acc

            @pl.when(c == nt - 1)
            def _():
                out[...] = jnp.maximum(out[...] + acc + b_ref[...], 0.0)

    return pl.pallas_call(body, grid=grid, in_specs=in_specs,
                          out_specs=out_specs, out_shape=out_shape)(*ts, wt, bias)


def _layer(xt, dis_t, wt, bias, src, dst, nt, dout, tiled_out):
    prop = _prop3 if nt == 3 else _prop2
    merge = nt == 3
    g0 = _rec(xt, None, dis_t, 1.0, False, nt, False)
    r1 = prop(g0, src, dst)
    tx1, g1 = _rec(r1, None, dis_t, -1.0, True, nt, merge)
    r2 = prop(g1, src, dst)
    tx2, g2 = _rec(r2, xt, dis_t, -2.0, True, nt, merge)
    r3 = prop(g2, src, dst)
    tx3, g3 = _rec(r3, tx1, dis_t, -2.0, True, nt, merge)
    r4 = prop(g3, src, dst)
    tx4 = _rec(r4, tx2, dis_t, -2.0, False, nt, merge)
    return _cheb_out([xt, tx1, tx2, tx3, tx4], wt, bias, nt, dout, tiled_out)


def kernel(x, edge_index, W1, b1, Wl1, bl1, W2, b2, Wl2, bl2):
    pad_e = _N + jnp.arange(_EP - _E, dtype=jnp.int32) % (_NP - _N)
    srcf = jnp.concatenate([edge_index[0], pad_e])
    dstf = jnp.concatenate([edge_index[1], pad_e])
    src_t3 = jnp.concatenate([srcf + t * _NP for t in range(3)])
    src_t2 = src_t3[:2 * _EP]

    parts = _deg_hist(srcf.reshape(_NB, _EB))
    deg = parts[:_N, 0] + parts[_NP:_NP + _N, 0]
    dis = jnp.where(deg > 0, deg ** -0.5, 0.0)
    dis_p = jnp.pad(dis, (0, _NP - _N))[:, None]

    x_pad = jnp.pad(x, ((0, _NP - _N), (0, 384 - 350)))
    xt1 = jnp.concatenate(
        [x_pad[:, :128], x_pad[:, 128:256], x_pad[:, 256:]], axis=0)
    dis3 = jnp.concatenate([dis_p, dis_p, dis_p], axis=0)

    w1 = jnp.pad(W1, ((0, 0), (0, 34), (0, 0)))
    w1 = w1.at[0].add(jnp.pad(Wl1, ((0, 34), (0, 0))))
    wt1 = w1.reshape(_K, 3, _DT, 256)
    bb1 = (b1 + bl1)[None, :]

    h_t = _layer(xt1, dis3, wt1, bb1, src_t3, dstf, 3, 256, True)
    xt2 = h_t.reshape(2 * _NP, _DT)
    dis2 = jnp.concatenate([dis_p, dis_p], axis=0)

    w2 = W2.at[0].add(Wl2)
    wt2 = w2.reshape(_K, 2, _DT, _DT)
    bb2 = (b2 + bl2)[None, :]

    out = _layer(xt2, dis2, wt2, bb2, src_t2, dstf, 2, _DT, False)
    return out[:_N]


# speedup vs baseline: 2.4337x; 1.6022x over previous
import functools

import jax
import jax.numpy as jnp
from jax import lax
from jax.experimental import pallas as pl
from jax.experimental.pallas import tpu as pltpu
from jax.experimental.pallas import tpu_sc as plsc

_N = 10000
_NP = 10240
_E = 160000
_K = 5
_DT = 128
_NS = 16
_EB = 128
_NB = 1280
_EP = _NB * _EB
_BSUB = _NB // _NS
_N_SUB = _NP // _NS
_ZR = 64


def _sc_mesh():
    return plsc.VectorSubcoreMesh(core_axis_name="c", subcore_axis_name="s")


def _zero_fill(zero_v):
    @pl.loop(0, _ZR)
    def _(rr):
        @pl.loop(0, _DT, step=16)
        def _(j):
            zero_v[rr, pl.ds(j, 16)] = jnp.zeros((16,), jnp.float32)


def _zero_acc(acc_sh, zero_v, s):
    @pl.loop(0, _N_SUB // _ZR)
    def _(zz):
        pltpu.sync_copy(zero_v, acc_sh.at[pl.ds(s * _N_SUB + zz * _ZR, _ZR)])


def _stream_batches(g_hbm, acc_sh, srct_hbm, dst_hbm, srci_v, dsti_v,
                    rows, isems, dsems, gsems, src_row, dst_row, n):
    def issue_idx(k, b):
        pltpu.async_copy(srct_hbm.at[src_row(k)],
                         srci_v.at[pl.ds(b, 1)], isems[b])
        pltpu.async_copy(dst_hbm.at[dst_row(k)],
                         dsti_v.at[pl.ds(b, 1)], dsems[b])

    def wait_isrc(k, b):
        pltpu.make_async_copy(srct_hbm.at[src_row(k)],
                              srci_v.at[pl.ds(b, 1)], isems[b]).wait()

    def wait_idst(k, b):
        pltpu.make_async_copy(dst_hbm.at[dst_row(k)],
                              dsti_v.at[pl.ds(b, 1)], dsems[b]).wait()

    def issue_gather(b):
        pltpu.async_copy(g_hbm.at[srci_v.at[b]], rows[b], gsems[b])

    def wait_gather(b):
        pltpu.make_async_copy(g_hbm.at[srci_v.at[b]], rows[b],
                              gsems[b]).wait()

    issue_idx(0, 0)
    issue_idx(1, 1)
    wait_isrc(0, 0)
    issue_gather(0)

    @pl.loop(0, n // 2)
    def _(io):
        for b in range(2):
            k = 2 * io + b
            wait_gather(b)

            @pl.when(k + 1 < n)
            def _():
                wait_isrc(k + 1, 1 - b)
                issue_gather(1 - b)

            wait_idst(k, b)
            pltpu.sync_copy(rows[b], acc_sh.at[dsti_v.at[b]], add=True)

            @pl.when(k + 2 < n)
            def _():
                issue_idx(k + 2, b)


def _prop_scratch():
    return [
        pltpu.VMEM((2, _EB), jnp.int32),
        pltpu.VMEM((2, _EB), jnp.int32),
        pltpu.VMEM((_EB, _DT), jnp.float32),
        pltpu.VMEM((_EB, _DT), jnp.float32),
        pltpu.VMEM((_ZR, _DT), jnp.float32),
        pltpu.VMEM_SHARED((_NP, _DT), jnp.float32),
    ] + [pltpu.SemaphoreType.DMA] * 6


@functools.partial(
    pl.kernel,
    out_type=jax.ShapeDtypeStruct((2 * _NP, _DT), jnp.float32),
    mesh=_sc_mesh(),
    scratch_types=_prop_scratch(),
)
def _prop2(g_hbm, srct_hbm, dst_hbm, out_hbm,
           srci_v, dsti_v, rows0, rows1, zero_v, acc_sh,
           is0, is1, ds0, ds1, gs0, gs1):
    c = lax.axis_index("c")
    s = lax.axis_index("s")
    _zero_fill(zero_v)
    _zero_acc(acc_sh, zero_v, s)
    plsc.subcore_barrier()
    _stream_batches(g_hbm, acc_sh, srct_hbm, dst_hbm, srci_v, dsti_v,
                    (rows0, rows1), (is0, is1), (ds0, ds1), (gs0, gs1),
                    lambda k: c * _NB + s * _BSUB + k,
                    lambda k: s * _BSUB + k, _BSUB)
    plsc.subcore_barrier()
    pltpu.sync_copy(acc_sh.at[pl.ds(s * _N_SUB, _N_SUB)],
                    out_hbm.at[pl.ds(c * _NP + s * _N_SUB, _N_SUB)])


@functools.partial(
    pl.kernel,
    out_type=jax.ShapeDtypeStruct((4 * _NP, _DT), jnp.float32),
    mesh=_sc_mesh(),
    scratch_types=_prop_scratch(),
)
def _prop3(g_hbm, srct_hbm, dst_hbm, out_hbm,
           srci_v, dsti_v, rows0, rows1, zero_v, acc_sh,
           is0, is1, ds0, ds1, gs0, gs1):
    c = lax.axis_index("c")
    s = lax.axis_index("s")
    _zero_fill(zero_v)

    _zero_acc(acc_sh, zero_v, s)
    plsc.subcore_barrier()
    _stream_batches(g_hbm, acc_sh, srct_hbm, dst_hbm, srci_v, dsti_v,
                    (rows0, rows1), (is0, is1), (ds0, ds1), (gs0, gs1),
                    lambda k: c * _NB + s * _BSUB + k,
                    lambda k: s * _BSUB + k, _BSUB)
    plsc.subcore_barrier()
    pltpu.sync_copy(acc_sh.at[pl.ds(s * _N_SUB, _N_SUB)],
                    out_hbm.at[pl.ds(c * _NP + s * _N_SUB, _N_SUB)])

    plsc.subcore_barrier()
    _zero_acc(acc_sh, zero_v, s)
    plsc.subcore_barrier()
    _stream_batches(g_hbm, acc_sh, srct_hbm, dst_hbm, srci_v, dsti_v,
                    (rows0, rows1), (is0, is1), (ds0, ds1), (gs0, gs1),
                    lambda k: 2 * _NB + s * _BSUB + 2 * k + c,
                    lambda k: s * _BSUB + 2 * k + c, _BSUB // 2)
    plsc.subcore_barrier()
    pltpu.sync_copy(acc_sh.at[pl.ds(s * _N_SUB, _N_SUB)],
                    out_hbm.at[pl.ds((2 + c) * _NP + s * _N_SUB, _N_SUB)])


@functools.partial(
    pl.kernel,
    out_type=jax.ShapeDtypeStruct((2 * _NP, _DT), jnp.float32),
    mesh=_sc_mesh(),
    scratch_types=[
        pltpu.VMEM((_BSUB, _EB), jnp.int32),
        pltpu.VMEM((_EB, _DT), jnp.float32),
        pltpu.VMEM((_ZR, _DT), jnp.float32),
        pltpu.VMEM_SHARED((_NP, _DT), jnp.float32),
    ],
)
def _deg_hist(src_hbm, out_hbm, src2_v, ones_v, zero_v, acc_sh):
    c = lax.axis_index("c")
    s = lax.axis_index("s")
    pltpu.sync_copy(src_hbm.at[pl.ds(s * _BSUB, _BSUB)], src2_v)

    @pl.loop(0, _EB)
    def _(rr):
        @pl.loop(0, _DT, step=16)
        def _(j):
            ones_v[rr, pl.ds(j, 16)] = jnp.ones((16,), jnp.float32)

    _zero_fill(zero_v)
    _zero_acc(acc_sh, zero_v, s)
    plsc.subcore_barrier()

    @pl.loop(0, _BSUB // 2)
    def _(i):
        pltpu.sync_copy(ones_v, acc_sh.at[src2_v.at[2 * i + c]], add=True)

    plsc.subcore_barrier()
    pltpu.sync_copy(acc_sh.at[pl.ds(s * _N_SUB, _N_SUB)],
                    out_hbm.at[pl.ds(c * _NP + s * _N_SUB, _N_SUB)])


_BM = 512


def _rec(r, prev, dis_t, a2, with_g, nt, merge):
    nb = _NP // _BM
    grid = (nt * nb,)
    rows = nt * _NP
    ins = [r]
    in_specs = [pl.BlockSpec((_BM, _DT), lambda i: (i, 0))]
    if merge:
        ins.append(r)
        in_specs.append(pl.BlockSpec((_BM, _DT), lambda i: (i + nb, 0)))
    if prev is not None:
        ins.append(prev)
        in_specs.append(pl.BlockSpec((_BM, _DT), lambda i: (i, 0)))
    ins.append(dis_t)
    in_specs.append(pl.BlockSpec((_BM, 1), lambda i: (i, 0)))
    n_out = 2 if with_g else 1
    out_shape = [jax.ShapeDtypeStruct((rows, _DT), jnp.float32)] * n_out
    out_specs = [pl.BlockSpec((_BM, _DT), lambda i: (i, 0))] * n_out

    def body(*refs):
        k = 0
        r_ref = refs[k]; k += 1
        rb_ref = None
        if merge:
            rb_ref = refs[k]; k += 1
        p_ref = None
        if prev is not None:
            p_ref = refs[k]; k += 1
        d_ref = refs[k]; k += 1
        outs = refs[k:]
        rv = r_ref[...]
        if merge:
            in_last = pl.program_id(0) >= (nt - 1) * nb
            rv = rv + jnp.where(in_last, rb_ref[...], 0.0)
        tx = (a2 * d_ref[...]) * rv
        if prev is not None:
            tx = tx - p_ref[...]
        outs[0][...] = tx
        if with_g:
            outs[1][...] = d_ref[...] * tx

    res = pl.pallas_call(body, grid=grid, in_specs=in_specs,
                         out_specs=out_specs, out_shape=out_shape)(*ins)
    return res if with_g else res[0]


def _cheb_out(ts, wt, bias, nt, dout, tiled_out):
    nb = _NP // _BM
    grid = (nb, nt)
    in_specs = [pl.BlockSpec((_BM, _DT), lambda i, c: (c * nb + i, 0))
                for _ in ts]
    in_specs.append(pl.BlockSpec((_K, 1, _DT, dout), lambda i, c: (0, c, 0, 0)))
    in_specs.append(pl.BlockSpec((1, dout), lambda i, c: (0, 0)))
    if tiled_out:
        hd = dout // 2
        out_shape = jax.ShapeDtypeStruct((2, _NP, hd), jnp.float32)
        out_specs = pl.BlockSpec((2, _BM, hd), lambda i, c: (0, i, 0))
    else:
        out_shape = jax.ShapeDtypeStruct((_NP, dout), jnp.float32)
        out_specs = pl.BlockSpec((_BM, dout), lambda i, c: (i, 0))

    def body(*refs):
        t_refs = refs[:_K]
        w_ref, b_ref, out = refs[_K], refs[_K + 1], refs[_K + 2]
        c = pl.program_id(1)
        acc = jnp.dot(t_refs[0][...], w_ref[0, 0],
                      preferred_element_type=jnp.float32)
        for k in range(1, _K):
            acc = acc + jnp.dot(t_refs[k][...], w_ref[k, 0],
                                preferred_element_type=jnp.float32)
        if tiled_out:
            hd2 = dout // 2

            @pl.when(c == 0)
            def _():
                out[0, ...] = acc[:, :hd2]
                out[1, ...] = acc[:, hd2:]

            @pl.when(jnp.logical_and(c > 0, c < nt - 1))
            def _():
                out[0, ...] = out[0, ...] + acc[:, :hd2]
                out[1, ...] = out[1, ...] + acc[:, hd2:]

            @pl.when(c == nt - 1)
            def _():
                out[0, ...] = jnp.maximum(
                    out[0, ...] + acc[:, :hd2] + b_ref[:, :hd2], 0.0)
                out[1, ...] = jnp.maximum(
                    out[1, ...] + acc[:, hd2:] + b_ref[:, hd2:], 0.0)
        else:
            @pl.when(c == 0)
            def _():
                out[...] = acc

            @pl.when(jnp.logical_and(c > 0, c < nt - 1))
            def _():
                out[...] = out[...] + acc

            @pl.when(c == nt - 1)
            def _():
                out[...] = jnp.maximum(out[...] + acc + b_ref[...], 0.0)

    return pl.pallas_call(body, grid=grid, in_specs=in_specs,
                          out_specs=out_specs, out_shape=out_shape)(*ts, wt, bias)


def _layer(xt, dis_t, wt, bias, src, dst, nt, dout, tiled_out):
    prop = _prop3 if nt == 3 else _prop2
    merge = nt == 3
    g0 = _rec(xt, None, dis_t, 1.0, False, nt, False)
    r1 = prop(g0, src, dst)
    tx1, g1 = _rec(r1, None, dis_t, -1.0, True, nt, merge)
    r2 = prop(g1, src, dst)
    tx2, g2 = _rec(r2, xt, dis_t, -2.0, True, nt, merge)
    r3 = prop(g2, src, dst)
    tx3, g3 = _rec(r3, tx1, dis_t, -2.0, True, nt, merge)
    r4 = prop(g3, src, dst)
    tx4 = _rec(r4, tx2, dis_t, -2.0, False, nt, merge)
    return _cheb_out([xt, tx1, tx2, tx3, tx4], wt, bias, nt, dout, tiled_out)


def kernel(x, edge_index, W1, b1, Wl1, bl1, W2, b2, Wl2, bl2):
    pad_e = _N + jnp.arange(_EP - _E, dtype=jnp.int32) % (_NP - _N)
    srcp = jnp.concatenate([edge_index[0], pad_e]).reshape(_NB, _EB)
    dstp = jnp.concatenate([edge_index[1], pad_e]).reshape(_NB, _EB)
    dst3d = dstp.reshape(_NB, 1, _EB)
    src_t3 = jnp.concatenate(
        [srcp + t * _NP for t in range(3)]).reshape(3 * _NB, 1, _EB)
    src_t2 = src_t3[:2 * _NB]

    parts = _deg_hist(srcp)
    deg = parts[:_N, 0] + parts[_NP:_NP + _N, 0]
    dis = jnp.where(deg > 0, deg ** -0.5, 0.0)
    dis_p = jnp.pad(dis, (0, _NP - _N))[:, None]

    x_pad = jnp.pad(x, ((0, _NP - _N), (0, 384 - 350)))
    xt1 = jnp.concatenate(
        [x_pad[:, :128], x_pad[:, 128:256], x_pad[:, 256:]], axis=0)
    dis3 = jnp.concatenate([dis_p, dis_p, dis_p], axis=0)

    w1 = jnp.pad(W1, ((0, 0), (0, 34), (0, 0)))
    w1 = w1.at[0].add(jnp.pad(Wl1, ((0, 34), (0, 0))))
    wt1 = w1.reshape(_K, 3, _DT, 256)
    bb1 = (b1 + bl1)[None, :]

    h_t = _layer(xt1, dis3, wt1, bb1, src_t3, dst3d, 3, 256, True)
    xt2 = h_t.reshape(2 * _NP, _DT)
    dis2 = jnp.concatenate([dis_p, dis_p], axis=0)

    w2 = W2.at[0].add(Wl2)
    wt2 = w2.reshape(_K, 2, _DT, _DT)
    bb2 = (b2 + bl2)[None, :]

    out = _layer(xt2, dis2, wt2, bb2, src_t2, dst3d, 2, _DT, False)
    return out[:_N]
